# baseline (speedup 1.0000x reference)
import jax
import jax.numpy as jnp
from jax import lax
from jax.experimental import pallas as pl
from jax.experimental.pallas import tpu as pltpu
from jax.experimental.pallas import tpu_sc as plsc

NC, NS, L = 2, 16, 16
NW = NC * NS

B, T, H, D = 4, 4096, 8, 64
N = B * T * H
IDX_MINOR = 128
ROWS_N = N // IDX_MINOR
ROWS_W = ROWS_N // NW
CHUNK_ROWS = 8
N_CHUNKS = ROWS_W // CHUNK_ROWS

TAB_N = 800532
TAB_PAD = 801792
BLK_C = 2304
GRID = TAB_PAD // BLK_C
BLK_A = BLK_C // 16
PAIRS = BLK_C // 256


def _detile_body(t_ref, o_ref):
    x = t_ref[...]
    for p in range(PAIRS):
        z = jnp.concatenate(
            [x[:, 256 * p : 256 * p + 128], x[:, 256 * p + 128 : 256 * p + 256]],
            axis=0,
        )
        o_ref[pl.ds(16 * p, 16)] = z.T.reshape(16, 8, 128)


def _detile(table_t):
    return pl.pallas_call(
        _detile_body,
        grid=(GRID,),
        in_specs=[pl.BlockSpec((D, BLK_C), lambda j: (0, j))],
        out_specs=pl.BlockSpec((BLK_A, 8, 2 * D), lambda j: (j, 0, 0)),
        out_shape=jax.ShapeDtypeStruct((TAB_PAD // 16, 8, 2 * D), jnp.float32),
        compiler_params=pltpu.CompilerParams(
            dimension_semantics=("parallel",),
        ),
    )(table_t)


def _gather_body(ids_hbm, table_hbm, off_hbm, out_hbm, idx_v, rows_v, off_v, sem):
    wid = lax.axis_index("c") * NS + lax.axis_index("s")
    row0 = wid * ROWS_W

    pltpu.sync_copy(ids_hbm.at[pl.ds(row0, ROWS_W)], idx_v)

    pltpu.sync_copy(off_hbm, off_v.at[pl.ds(0, H)])
    pltpu.sync_copy(off_hbm, off_v.at[pl.ds(H, H)])
    off16 = off_v[...]

    for r in range(ROWS_W):
        for k in range(IDX_MINOR // L):
            v = idx_v[r, pl.ds(k * L, L)] + off16
            m = v & 255
            idx_v[r, pl.ds(k * L, L)] = (v - m) + ((m & 127) << 1) + (m >> 7)

    for c in range(N_CHUNKS):
        copies = [
            pltpu.async_copy(
                table_hbm.at[idx_v.at[c * CHUNK_ROWS + j]],
                rows_v.at[pl.ds(j * IDX_MINOR, IDX_MINOR)],
                sem,
            )
            for j in range(CHUNK_ROWS)
        ]
        for cp in copies:
            cp.wait()
        out_row0 = (row0 + c * CHUNK_ROWS) * IDX_MINOR
        pltpu.sync_copy(
            rows_v,
            out_hbm.at[pl.ds(out_row0, CHUNK_ROWS * IDX_MINOR)],
        )


def _engram_gather(ids, table_lin, offsets):
    grid_kernel = pl.kernel(
        _gather_body,
        out_type=jax.ShapeDtypeStruct((N, D), jnp.float32),
        mesh=plsc.VectorSubcoreMesh(core_axis_name="c", subcore_axis_name="s"),
        scratch_types=[
            pltpu.VMEM((ROWS_W, IDX_MINOR), jnp.int32),
            pltpu.VMEM((CHUNK_ROWS * IDX_MINOR, D), jnp.float32),
            pltpu.VMEM((L,), jnp.int32),
            pltpu.SemaphoreType.DMA,
        ],
        compiler_params=pltpu.CompilerParams(use_tc_tiling_on_sc=False),
    )
    return grid_kernel(ids, table_lin, offsets)


@jax.jit
def _run(input_ids, table, offsets):
    ids = input_ids.astype(jnp.int32).reshape(ROWS_N, IDX_MINOR)
    table_lin = _detile(table.T).reshape(TAB_PAD, D)
    out = _engram_gather(ids, table_lin, offsets.astype(jnp.int32))
    return out.reshape(B, T, H, D)


def kernel(input_ids, table, offsets):
    return _run(input_ids, table, offsets)


# speedup vs baseline: 1.5786x; 1.5786x over previous
import jax
import jax.numpy as jnp
from jax import lax
from jax.experimental import pallas as pl
from jax.experimental.pallas import tpu as pltpu
from jax.experimental.pallas import tpu_sc as plsc

NC, NS, L = 2, 16, 16
NW = NC * NS

B, T, H, D = 4, 4096, 8, 64
N = B * T * H
IDX_MINOR = 128
ROWS_N = N // IDX_MINOR
ROWS_W = ROWS_N // NW
CHUNK_ROWS = 8
N_CHUNKS = ROWS_W // CHUNK_ROWS

TAB_N = 800532
TAB_PAD = 801792
BLK_C = 27648
GRID = TAB_PAD // BLK_C
BLK_A = BLK_C // 16
PAIRS = BLK_C // 256


def _detile_body(t_ref, o_ref):
    x = t_ref[...]
    for p in range(PAIRS):
        z = jnp.concatenate(
            [x[:, 256 * p : 256 * p + 128], x[:, 256 * p + 128 : 256 * p + 256]],
            axis=0,
        )
        o_ref[pl.ds(16 * p, 16)] = z.T.reshape(16, 8, 128)


def _detile(table_t):
    return pl.pallas_call(
        _detile_body,
        grid=(GRID,),
        in_specs=[pl.BlockSpec((D, BLK_C), lambda j: (0, j))],
        out_specs=pl.BlockSpec((BLK_A, 8, 2 * D), lambda j: (j, 0, 0)),
        out_shape=jax.ShapeDtypeStruct((TAB_PAD // 16, 8, 2 * D), jnp.float32),
        compiler_params=pltpu.CompilerParams(
            dimension_semantics=("parallel",),
        ),
    )(table_t)


def _gather_body(ids_hbm, table_hbm, off_hbm, out_hbm, idx_v, rows_v, off_v, sem):
    wid = lax.axis_index("c") * NS + lax.axis_index("s")
    row0 = wid * ROWS_W

    pltpu.sync_copy(ids_hbm.at[pl.ds(row0, ROWS_W)], idx_v)

    pltpu.sync_copy(off_hbm, off_v.at[pl.ds(0, H)])
    pltpu.sync_copy(off_hbm, off_v.at[pl.ds(H, H)])
    off16 = off_v[...]

    for r in range(ROWS_W):
        for k in range(IDX_MINOR // L):
            v = idx_v[r, pl.ds(k * L, L)] + off16
            m = v & 255
            idx_v[r, pl.ds(k * L, L)] = (v - m) + ((m & 127) << 1) + (m >> 7)

    for c in range(N_CHUNKS):
        copies = [
            pltpu.async_copy(
                table_hbm.at[idx_v.at[c * CHUNK_ROWS + j]],
                rows_v.at[pl.ds(j * IDX_MINOR, IDX_MINOR)],
                sem,
            )
            for j in range(CHUNK_ROWS)
        ]
        for cp in copies:
            cp.wait()
        out_row0 = (row0 + c * CHUNK_ROWS) * IDX_MINOR
        pltpu.sync_copy(
            rows_v,
            out_hbm.at[pl.ds(out_row0, CHUNK_ROWS * IDX_MINOR)],
        )


def _engram_gather(ids, table_lin, offsets):
    grid_kernel = pl.kernel(
        _gather_body,
        out_type=jax.ShapeDtypeStruct((N, D), jnp.float32),
        mesh=plsc.VectorSubcoreMesh(core_axis_name="c", subcore_axis_name="s"),
        scratch_types=[
            pltpu.VMEM((ROWS_W, IDX_MINOR), jnp.int32),
            pltpu.VMEM((CHUNK_ROWS * IDX_MINOR, D), jnp.float32),
            pltpu.VMEM((L,), jnp.int32),
            pltpu.SemaphoreType.DMA,
        ],
        compiler_params=pltpu.CompilerParams(use_tc_tiling_on_sc=False),
    )
    return grid_kernel(ids, table_lin, offsets)


@jax.jit
def _run(input_ids, table, offsets):
    ids = input_ids.astype(jnp.int32).reshape(ROWS_N, IDX_MINOR)
    table_lin = _detile(table.T).reshape(TAB_PAD, D)
    out = _engram_gather(ids, table_lin, offsets.astype(jnp.int32))
    return out.reshape(B, T, H, D)


def kernel(input_ids, table, offsets):
    return _run(input_ids, table, offsets)


# speedup vs baseline: 1.6600x; 1.0516x over previous
import jax
import jax.numpy as jnp
from jax import lax
from jax.experimental import pallas as pl
from jax.experimental.pallas import tpu as pltpu
from jax.experimental.pallas import tpu_sc as plsc

NC, NS, L = 2, 16, 16
NW = NC * NS

B, T, H, D = 4, 4096, 8, 64
N = B * T * H
IDX_MINOR = 128
ROWS_N = N // IDX_MINOR
ROWS_W = ROWS_N // NW
CHUNK_ROWS = 8
N_CHUNKS = ROWS_W // CHUNK_ROWS

TAB_N = 800532
TAB_PAD = 801792
BLK_C = 27648
GRID = TAB_PAD // BLK_C
BLK_A = BLK_C // 16
PAIRS = BLK_C // 256


def _detile_body(t_ref, o_ref):
    x = t_ref[...]
    for p in range(PAIRS):
        z = jnp.concatenate(
            [x[:, 256 * p : 256 * p + 128], x[:, 256 * p + 128 : 256 * p + 256]],
            axis=0,
        )
        o_ref[pl.ds(16 * p, 16)] = z.T.reshape(16, 8, 128)


def _detile(table_t):
    return pl.pallas_call(
        _detile_body,
        grid=(GRID,),
        in_specs=[pl.BlockSpec((D, BLK_C), lambda j: (0, j))],
        out_specs=pl.BlockSpec((BLK_A, 8, 2 * D), lambda j: (j, 0, 0)),
        out_shape=jax.ShapeDtypeStruct((TAB_PAD // 16, 8, 2 * D), jnp.float32),
        compiler_params=pltpu.CompilerParams(
            dimension_semantics=("parallel",),
        ),
    )(table_t)


TT = T // IDX_MINOR
N_PAIRS = B * TT
PAIRS_W = N_PAIRS // NW


def _gather_body(ids_hbm, table_hbm, off_hbm, out_hbm, idx_v, rows_v, off_v, sem):
    wid = lax.axis_index("c") * NS + lax.axis_index("s")

    pltpu.sync_copy(off_hbm, off_v)

    for pp in range(PAIRS_W):
        p = wid * PAIRS_W + pp
        b = p // TT
        tt = p % TT

        pltpu.sync_copy(ids_hbm.at[b, tt], idx_v)

        for h in range(H):
            bc = off_v[h]
            for k in range(IDX_MINOR // L):
                v = idx_v[h, pl.ds(k * L, L)] + bc
                m = v & 255
                idx_v[h, pl.ds(k * L, L)] = (v - m) + ((m & 127) << 1) + (m >> 7)

        copies = [
            pltpu.async_copy(table_hbm.at[idx_v.at[h]], rows_v.at[h], sem)
            for h in range(H)
        ]
        for cp in copies:
            cp.wait()
        pltpu.sync_copy(rows_v, out_hbm.at[b, :, pl.ds(tt * IDX_MINOR, IDX_MINOR)])


def _engram_gather(ids, table_lin, offsets):
    grid_kernel = pl.kernel(
        _gather_body,
        out_type=jax.ShapeDtypeStruct((B, H, T, D), jnp.float32),
        mesh=plsc.VectorSubcoreMesh(core_axis_name="c", subcore_axis_name="s"),
        scratch_types=[
            pltpu.VMEM((H, IDX_MINOR), jnp.int32),
            pltpu.VMEM((H, IDX_MINOR, D), jnp.float32),
            pltpu.VMEM((H, L), jnp.int32),
            pltpu.SemaphoreType.DMA,
        ],
        compiler_params=pltpu.CompilerParams(use_tc_tiling_on_sc=False),
    )
    return grid_kernel(ids, table_lin, offsets)


@jax.jit
def _run(input_ids, table, offsets):
    ids = (
        input_ids.astype(jnp.int32)
        .transpose(0, 2, 1)
        .reshape(B, H, TT, IDX_MINOR)
        .transpose(0, 2, 1, 3)
    )
    off_b = jnp.repeat(offsets.astype(jnp.int32)[:, None], L, axis=1)
    table_lin = _detile(table.T).reshape(TAB_PAD, D)
    out = _engram_gather(ids, table_lin, off_b)
    return out.transpose(0, 2, 1, 3)


def kernel(input_ids, table, offsets):
    return _run(input_ids, table, offsets)


# speedup vs baseline: 1.7298x; 1.0420x over previous
import jax
import jax.numpy as jnp
from jax import lax
from jax.experimental import pallas as pl
from jax.experimental.pallas import tpu as pltpu
from jax.experimental.pallas import tpu_sc as plsc

NC, NS, L = 2, 16, 16
NW = NC * NS

B, T, H, D = 4, 4096, 8, 64
N = B * T * H
IDX_MINOR = 128
ROWS_N = N // IDX_MINOR
ROWS_W = ROWS_N // NW
CHUNK_ROWS = 8
N_CHUNKS = ROWS_W // CHUNK_ROWS

TAB_N = 800532
TAB_PAD = 801792
BLK_C = 27648
GRID = TAB_PAD // BLK_C
BLK_A = BLK_C // 16
PAIRS = BLK_C // 256


def _detile_body(t_ref, o_ref):
    x = t_ref[...]
    for p in range(PAIRS):
        z = jnp.concatenate(
            [x[:, 256 * p : 256 * p + 128], x[:, 256 * p + 128 : 256 * p + 256]],
            axis=0,
        )
        o_ref[pl.ds(16 * p, 16)] = z.T.reshape(16, 8, 128)


def _detile(table_t):
    return pl.pallas_call(
        _detile_body,
        grid=(GRID,),
        in_specs=[pl.BlockSpec((D, BLK_C), lambda j: (0, j))],
        out_specs=pl.BlockSpec((BLK_A, 8, 2 * D), lambda j: (j, 0, 0)),
        out_shape=jax.ShapeDtypeStruct((TAB_PAD // 16, 8, 2 * D), jnp.float32),
        compiler_params=pltpu.CompilerParams(
            dimension_semantics=("parallel",),
        ),
    )(table_t)


TT = T // IDX_MINOR
N_PAIRS = B * TT
PAIRS_W = N_PAIRS // NW


def _gather_body(ids_hbm, table_hbm, off_hbm, out_hbm, idx_v, rows_v, off_v, sem):
    wid = lax.axis_index("c") * NS + lax.axis_index("s")

    pltpu.sync_copy(off_hbm, off_v)

    for pp in range(PAIRS_W):
        p = wid * PAIRS_W + pp
        b = p // TT
        tt = p % TT

        pltpu.sync_copy(ids_hbm.at[b, tt], idx_v)

        for h in range(H):
            bc = off_v[h]
            for k in range(IDX_MINOR // L):
                v = idx_v[h, pl.ds(k * L, L)] + bc
                m = v & 255
                idx_v[h, pl.ds(k * L, L)] = (v - m) + ((m & 127) << 1) + (m >> 7)

        copies = [
            pltpu.async_copy(table_hbm.at[idx_v.at[h]], rows_v.at[h], sem)
            for h in range(H)
        ]
        for cp in copies:
            cp.wait()
        pltpu.sync_copy(
            rows_v,
            out_hbm.at[b, :, pl.ds(tt * IDX_MINOR, IDX_MINOR), pl.ds(0, D)],
        )


def _engram_gather(ids, table_lin, offsets):
    grid_kernel = pl.kernel(
        _gather_body,
        out_type=jax.ShapeDtypeStruct((B, H, T, 2 * D), jnp.float32),
        mesh=plsc.VectorSubcoreMesh(core_axis_name="c", subcore_axis_name="s"),
        scratch_types=[
            pltpu.VMEM((H, IDX_MINOR), jnp.int32),
            pltpu.VMEM((H, IDX_MINOR, D), jnp.float32),
            pltpu.VMEM((H, L), jnp.int32),
            pltpu.SemaphoreType.DMA,
        ],
        compiler_params=pltpu.CompilerParams(use_tc_tiling_on_sc=False),
    )
    return grid_kernel(ids, table_lin, offsets)


@jax.jit
def _run(input_ids, table, offsets):
    ids = (
        input_ids.astype(jnp.int32)
        .transpose(0, 2, 1)
        .reshape(B, H, TT, IDX_MINOR)
        .transpose(0, 2, 1, 3)
    )
    off_b = jnp.repeat(offsets.astype(jnp.int32)[:, None], L, axis=1)
    table_lin = _detile(table.T).reshape(TAB_PAD, D)
    out = _engram_gather(ids, table_lin, off_b)
    return out[..., :D].transpose(0, 2, 1, 3)


def kernel(input_ids, table, offsets):
    return _run(input_ids, table, offsets)
